# baseline (speedup 1.0000x reference)
import functools

import jax
import jax.numpy as jnp
from jax import lax
from jax.experimental import pallas as pl
from jax.experimental.pallas import tpu as pltpu
from jax.experimental.pallas import tpu_sc as plsc

_VOCAB = 1000000
_DIM = 64
_BATCH = 4096
_HIST = 200

_B = _BATCH * _HIST
_NC = 2
_NS = 16
_NW = _NC * _NS
_B_PER_W = _B // _NW
_CHUNK = 512
_N_CHUNKS = _B_PER_W // _CHUNK
_NBUF = 2

_CW = 128
_NBLK = _VOCAB // _CW
_REM = _VOCAB - _NBLK * _CW

_mesh = plsc.VectorSubcoreMesh(core_axis_name="c", subcore_axis_name="s")


@functools.partial(
    pl.kernel,
    mesh=_mesh,
    out_type=jax.ShapeDtypeStruct((_VOCAB // 2, 2 * _DIM), jnp.float32),
    compiler_params=pltpu.CompilerParams(
        use_tc_tiling_on_sc=True,
        needs_layout_passes=False,
        disable_bounds_checks=True,
    ),
    scratch_types=[
        pltpu.VMEM((_DIM, _CW), jnp.float32),
        pltpu.VMEM((_CW // 2, 2 * _DIM), jnp.float32),
    ],
)
def _transpose_k(embt_hbm, out_hbm, inb, tb):
    wid = lax.axis_index("s") * _NC + lax.axis_index("c")
    d16 = lax.iota(jnp.int32, 16)

    def do_cols(ncols):
        @pl.loop(0, ncols)
        def col(j):
            row = j // 2
            cb = (j % 2) * _DIM
            jv = jnp.full((16,), j, jnp.int32)
            for k in range(_DIM // 16):
                vals = plsc.load_gather(inb, [d16 + 16 * k, jv])
                tb[row, pl.ds(cb + 16 * k, 16)] = vals

    @pl.loop(wid, _NBLK + 1, step=_NW)
    def blk(i):
        c0 = i * _CW
        pltpu.sync_copy(embt_hbm.at[:, pl.ds(c0, _CW)], inb)
        do_cols(_CW)

        @pl.when(i < _NBLK)
        def _():
            pltpu.sync_copy(tb, out_hbm.at[pl.ds(i * (_CW // 2), _CW // 2), :])

        @pl.when(i == _NBLK)
        def _():
            pltpu.sync_copy(
                tb.at[pl.ds(0, _REM // 2), :],
                out_hbm.at[pl.ds(_NBLK * (_CW // 2), _REM // 2), :],
            )


@functools.partial(
    pl.kernel,
    mesh=_mesh,
    out_type=jax.ShapeDtypeStruct((_B, 2 * _DIM), jnp.float32),
    compiler_params=pltpu.CompilerParams(use_tc_tiling_on_sc=False),
    scratch_types=[
        pltpu.VMEM((_B_PER_W,), jnp.int32),
        pltpu.VMEM((_NBUF, _CHUNK, _DIM), jnp.float32),
        pltpu.SemaphoreType.DMA,
        pltpu.SemaphoreType.DMA,
        pltpu.SemaphoreType.DMA,
        pltpu.SemaphoreType.DMA,
    ],
)
def _emb_lookup(idx_hbm, table_hbm, out_hbm, idx_v, rows_v, g0, g1, o0, o1):
    g_sems = [g0, g1]
    o_sems = [o0, o1]
    wid = lax.axis_index("s") * _NC + lax.axis_index("c")
    base = wid * _B_PER_W

    pltpu.sync_copy(idx_hbm.at[pl.ds(base, _B_PER_W)], idx_v)

    def gather_desc(i, b):
        return pltpu.make_async_copy(
            table_hbm.at[idx_v.at[pl.ds(i * _CHUNK, _CHUNK)]],
            rows_v.at[b],
            g_sems[b],
        )

    def out_desc(i, b):
        return pltpu.make_async_copy(
            rows_v.at[b],
            out_hbm.at[pl.ds(base + i * _CHUNK, _CHUNK), pl.ds(0, _DIM)],
            o_sems[b],
        )

    for b in range(_NBUF):
        gather_desc(b, b).start()

    @pl.loop(0, _N_CHUNKS, step=_NBUF)
    def body(g):
        for b in range(_NBUF):
            i = g + b
            gather_desc(i, b).wait()
            out_desc(i, b).start()
        for b in range(_NBUF):
            i = g + b + _NBUF

            @pl.when(i < _N_CHUNKS)
            def _():
                out_desc(i - _NBUF, b).wait()
                gather_desc(i, b).start()

    for b in range(_NBUF):
        out_desc(_N_CHUNKS - _NBUF + b, b).wait()


def kernel(x, emb_weight):
    pairs = _transpose_k(emb_weight.T)
    table_lin = pairs.reshape(_VOCAB, _DIM)
    flat = x.reshape(_B)
    out2 = _emb_lookup(flat, table_lin)
    return out2[:, :_DIM].reshape(_BATCH, _HIST, _DIM)


# speedup vs baseline: 1.1872x; 1.1872x over previous
import functools

import jax
import jax.numpy as jnp
from jax import lax
from jax.experimental import pallas as pl
from jax.experimental.pallas import tpu as pltpu
from jax.experimental.pallas import tpu_sc as plsc

_VOCAB = 1000000
_DIM = 64
_BATCH = 4096
_HIST = 200

_B = _BATCH * _HIST
_NC = 2
_NS = 16
_NW = _NC * _NS
_B_PER_W = _B // _NW
_CHUNK = 512
_N_CHUNKS = _B_PER_W // _CHUNK
_NBUF = 2

_CW = 128
_NBLK = _VOCAB // _CW

_mesh = plsc.VectorSubcoreMesh(core_axis_name="c", subcore_axis_name="s")


@functools.partial(
    pl.kernel,
    mesh=_mesh,
    out_type=jax.ShapeDtypeStruct((_VOCAB // 2, 2 * _DIM), jnp.float32),
    compiler_params=pltpu.CompilerParams(
        use_tc_tiling_on_sc=True,
        needs_layout_passes=False,
        disable_bounds_checks=True,
    ),
    scratch_types=[
        pltpu.VMEM((_NBUF, _DIM, _CW), jnp.float32),
        pltpu.VMEM((_NBUF, _CW // 2, 2 * _DIM), jnp.float32),
        pltpu.SemaphoreType.DMA,
        pltpu.SemaphoreType.DMA,
        pltpu.SemaphoreType.DMA,
        pltpu.SemaphoreType.DMA,
    ],
)
def _transpose_k(embt_hbm, out_hbm, inb, tb, i0, i1, o0, o1):
    i_sems = [i0, i1]
    o_sems = [o0, o1]
    wid = lax.axis_index("s") * _NC + lax.axis_index("c")
    d16k = [lax.iota(jnp.int32, 16) + 16 * k for k in range(_DIM // 16)]

    def in_desc(i, b):
        return pltpu.make_async_copy(
            embt_hbm.at[:, pl.ds(i * _CW, _CW)], inb.at[b], i_sems[b]
        )

    def out_desc(i, b):
        return pltpu.make_async_copy(
            tb.at[b], out_hbm.at[pl.ds(i * (_CW // 2), _CW // 2), :], o_sems[b]
        )

    def do_cols(b):
        inb_b = inb.at[b]
        tb_b = tb.at[b]

        @pl.loop(0, _CW // 2)
        def pair(r):
            jv0 = jnp.full((16,), 2 * r, jnp.int32)
            jv1 = jv0 + 1
            for k in range(_DIM // 16):
                tb_b[r, pl.ds(16 * k, 16)] = plsc.load_gather(inb_b, [d16k[k], jv0])
                tb_b[r, pl.ds(_DIM + 16 * k, 16)] = plsc.load_gather(
                    inb_b, [d16k[k], jv1]
                )

    _NT = (_NBLK + _NW - 1) // _NW
    _NSLOT = (_NT + 2 + 1) // 2 * 2

    for b in range(_NBUF):
        i = wid + b * _NW

        @pl.when(i < _NBLK)
        def _():
            in_desc(i, b).start()

    @pl.loop(0, _NSLOT, step=_NBUF)
    def body(g):
        for b in range(_NBUF):
            t = g + b
            i = wid + t * _NW
            prev = i - _NBUF * _NW

            @pl.when(jnp.logical_and(t >= _NBUF, prev < _NBLK))
            def _():
                out_desc(prev, b).wait()

            @pl.when(i < _NBLK)
            def _():
                in_desc(i, b).wait()
                do_cols(b)
                out_desc(i, b).start()
                nxt = i + _NBUF * _NW

                @pl.when(nxt < _NBLK)
                def _():
                    in_desc(nxt, b).start()

    @pl.when(wid == _NBLK % _NW)
    def _():
        c0t = _NBLK * _CW + wid * 0
        pltpu.sync_copy(embt_hbm.at[:, pl.ds(c0t, _CW)], inb.at[0])
        do_cols(0)
        pltpu.sync_copy(
            tb.at[0].at[pl.ds(0, _CW // 4), :],
            out_hbm.at[pl.ds(_NBLK * (_CW // 2), _CW // 4), :],
        )


@functools.partial(
    pl.kernel,
    mesh=_mesh,
    out_type=jax.ShapeDtypeStruct((_B, 2 * _DIM), jnp.float32),
    compiler_params=pltpu.CompilerParams(use_tc_tiling_on_sc=False),
    scratch_types=[
        pltpu.VMEM((_B_PER_W,), jnp.int32),
        pltpu.VMEM((_NBUF, _CHUNK, _DIM), jnp.float32),
        pltpu.SemaphoreType.DMA,
        pltpu.SemaphoreType.DMA,
        pltpu.SemaphoreType.DMA,
        pltpu.SemaphoreType.DMA,
    ],
)
def _emb_lookup(idx_hbm, table_hbm, out_hbm, idx_v, rows_v, g0, g1, o0, o1):
    g_sems = [g0, g1]
    o_sems = [o0, o1]
    wid = lax.axis_index("s") * _NC + lax.axis_index("c")
    base = wid * _B_PER_W

    pltpu.sync_copy(idx_hbm.at[pl.ds(base, _B_PER_W)], idx_v)

    def gather_desc(i, b):
        return pltpu.make_async_copy(
            table_hbm.at[idx_v.at[pl.ds(i * _CHUNK, _CHUNK)]],
            rows_v.at[b],
            g_sems[b],
        )

    def out_desc(i, b):
        return pltpu.make_async_copy(
            rows_v.at[b],
            out_hbm.at[pl.ds(base + i * _CHUNK, _CHUNK), pl.ds(0, _DIM)],
            o_sems[b],
        )

    for b in range(_NBUF):
        gather_desc(b, b).start()

    @pl.loop(0, _N_CHUNKS, step=_NBUF)
    def body(g):
        for b in range(_NBUF):
            i = g + b
            gather_desc(i, b).wait()
            out_desc(i, b).start()
        for b in range(_NBUF):
            i = g + b + _NBUF

            @pl.when(i < _N_CHUNKS)
            def _():
                out_desc(i - _NBUF, b).wait()
                gather_desc(i, b).start()

    for b in range(_NBUF):
        out_desc(_N_CHUNKS - _NBUF + b, b).wait()


def kernel(x, emb_weight):
    pairs = _transpose_k(emb_weight.T)
    table_lin = pairs.reshape(_VOCAB, _DIM)
    flat = x.reshape(_B)
    out2 = _emb_lookup(flat, table_lin)
    return out2[:, :_DIM].reshape(_BATCH, _HIST, _DIM)


# speedup vs baseline: 1.1883x; 1.0009x over previous
import functools

import jax
import jax.numpy as jnp
from jax import lax
from jax.experimental import pallas as pl
from jax.experimental.pallas import tpu as pltpu
from jax.experimental.pallas import tpu_sc as plsc

_VOCAB = 1000000
_DIM = 64
_BATCH = 4096
_HIST = 200

_B = _BATCH * _HIST
_NC = 2
_NS = 16
_NW = _NC * _NS
_B_PER_W = _B // _NW
_CHUNK = 512
_N_CHUNKS = _B_PER_W // _CHUNK
_NBUF = 2

_CW = 128
_NBLK = _VOCAB // _CW

_mesh = plsc.VectorSubcoreMesh(core_axis_name="c", subcore_axis_name="s")


@functools.partial(
    pl.kernel,
    mesh=_mesh,
    out_type=jax.ShapeDtypeStruct((_VOCAB // 2, 2 * _DIM), jnp.float32),
    compiler_params=pltpu.CompilerParams(
        use_tc_tiling_on_sc=True,
        needs_layout_passes=False,
        disable_bounds_checks=True,
    ),
    scratch_types=[
        pltpu.VMEM((_NBUF, _DIM, _CW), jnp.float32),
        pltpu.VMEM((_NBUF, _CW // 2, 2 * _DIM), jnp.float32),
        pltpu.SemaphoreType.DMA,
        pltpu.SemaphoreType.DMA,
        pltpu.SemaphoreType.DMA,
        pltpu.SemaphoreType.DMA,
    ],
)
def _transpose_k(embt_hbm, out_hbm, inb, tb, i0, i1, o0, o1):
    i_sems = [i0, i1]
    o_sems = [o0, o1]
    wid = lax.axis_index("s") * _NC + lax.axis_index("c")
    d16k = [lax.iota(jnp.int32, 16) + 16 * k for k in range(_DIM // 16)]

    def in_desc(i, b):
        return pltpu.make_async_copy(
            embt_hbm.at[:, pl.ds(i * _CW, _CW)], inb.at[b], i_sems[b]
        )

    def out_desc(i, b):
        return pltpu.make_async_copy(
            tb.at[b], out_hbm.at[pl.ds(i * (_CW // 2), _CW // 2), :], o_sems[b]
        )

    def do_cols(b):
        inb_b = inb.at[b]
        tb_b = tb.at[b]

        @pl.loop(0, _CW // 2, unroll=8)
        def pair(r):
            jv0 = jnp.full((16,), 2 * r, jnp.int32)
            jv1 = jv0 + 1
            for k in range(_DIM // 16):
                tb_b[r, pl.ds(16 * k, 16)] = plsc.load_gather(inb_b, [d16k[k], jv0])
                tb_b[r, pl.ds(_DIM + 16 * k, 16)] = plsc.load_gather(
                    inb_b, [d16k[k], jv1]
                )

    _NT = (_NBLK + _NW - 1) // _NW
    _NSLOT = (_NT + 2 + 1) // 2 * 2

    for b in range(_NBUF):
        i = wid + b * _NW

        @pl.when(i < _NBLK)
        def _():
            in_desc(i, b).start()

    @pl.loop(0, _NSLOT, step=_NBUF)
    def body(g):
        for b in range(_NBUF):
            t = g + b
            i = wid + t * _NW
            prev = i - _NBUF * _NW

            @pl.when(jnp.logical_and(t >= _NBUF, prev < _NBLK))
            def _():
                out_desc(prev, b).wait()

            @pl.when(i < _NBLK)
            def _():
                in_desc(i, b).wait()
                do_cols(b)
                out_desc(i, b).start()
                nxt = i + _NBUF * _NW

                @pl.when(nxt < _NBLK)
                def _():
                    in_desc(nxt, b).start()

    @pl.when(wid == _NBLK % _NW)
    def _():
        c0t = _NBLK * _CW + wid * 0
        pltpu.sync_copy(embt_hbm.at[:, pl.ds(c0t, _CW)], inb.at[0])
        do_cols(0)
        pltpu.sync_copy(
            tb.at[0].at[pl.ds(0, _CW // 4), :],
            out_hbm.at[pl.ds(_NBLK * (_CW // 2), _CW // 4), :],
        )


@functools.partial(
    pl.kernel,
    mesh=_mesh,
    out_type=jax.ShapeDtypeStruct((_B, 2 * _DIM), jnp.float32),
    compiler_params=pltpu.CompilerParams(use_tc_tiling_on_sc=False),
    scratch_types=[
        pltpu.VMEM((_B_PER_W,), jnp.int32),
        pltpu.VMEM((_NBUF, _CHUNK, _DIM), jnp.float32),
        pltpu.SemaphoreType.DMA,
        pltpu.SemaphoreType.DMA,
        pltpu.SemaphoreType.DMA,
        pltpu.SemaphoreType.DMA,
    ],
)
def _emb_lookup(idx_hbm, table_hbm, out_hbm, idx_v, rows_v, g0, g1, o0, o1):
    g_sems = [g0, g1]
    o_sems = [o0, o1]
    wid = lax.axis_index("s") * _NC + lax.axis_index("c")
    base = wid * _B_PER_W

    pltpu.sync_copy(idx_hbm.at[pl.ds(base, _B_PER_W)], idx_v)

    def gather_desc(i, b):
        return pltpu.make_async_copy(
            table_hbm.at[idx_v.at[pl.ds(i * _CHUNK, _CHUNK)]],
            rows_v.at[b],
            g_sems[b],
        )

    def out_desc(i, b):
        return pltpu.make_async_copy(
            rows_v.at[b],
            out_hbm.at[pl.ds(base + i * _CHUNK, _CHUNK), pl.ds(0, _DIM)],
            o_sems[b],
        )

    for b in range(_NBUF):
        gather_desc(b, b).start()

    @pl.loop(0, _N_CHUNKS, step=_NBUF)
    def body(g):
        for b in range(_NBUF):
            i = g + b
            gather_desc(i, b).wait()
            out_desc(i, b).start()
        for b in range(_NBUF):
            i = g + b + _NBUF

            @pl.when(i < _N_CHUNKS)
            def _():
                out_desc(i - _NBUF, b).wait()
                gather_desc(i, b).start()

    for b in range(_NBUF):
        out_desc(_N_CHUNKS - _NBUF + b, b).wait()


def kernel(x, emb_weight):
    pairs = _transpose_k(emb_weight.T)
    table_lin = pairs.reshape(_VOCAB, _DIM)
    flat = x.reshape(_B)
    out2 = _emb_lookup(flat, table_lin)
    return out2[:, :_DIM].reshape(_BATCH, _HIST, _DIM)


# speedup vs baseline: 1.1911x; 1.0024x over previous
import functools

import jax
import jax.numpy as jnp
from jax import lax
from jax.experimental import pallas as pl
from jax.experimental.pallas import tpu as pltpu
from jax.experimental.pallas import tpu_sc as plsc

_VOCAB = 1000000
_DIM = 64
_BATCH = 4096
_HIST = 200

_B = _BATCH * _HIST
_NC = 2
_NS = 16
_NW = _NC * _NS
_B_PER_W = _B // _NW
_CHUNK = 512
_N_CHUNKS = _B_PER_W // _CHUNK
_NBUF = 2

_CW = 128
_NBLK = _VOCAB // _CW

_mesh = plsc.VectorSubcoreMesh(core_axis_name="c", subcore_axis_name="s")


@functools.partial(
    pl.kernel,
    mesh=_mesh,
    out_type=jax.ShapeDtypeStruct((_VOCAB // 2, 2 * _DIM), jnp.float32),
    compiler_params=pltpu.CompilerParams(
        use_tc_tiling_on_sc=True,
        needs_layout_passes=False,
        disable_bounds_checks=True,
    ),
    scratch_types=[
        pltpu.VMEM((_NBUF, _DIM, _CW), jnp.float32),
        pltpu.VMEM((_NBUF, _CW // 2, 2 * _DIM), jnp.float32),
        pltpu.SemaphoreType.DMA,
        pltpu.SemaphoreType.DMA,
        pltpu.SemaphoreType.DMA,
        pltpu.SemaphoreType.DMA,
    ],
)
def _transpose_k(embt_hbm, out_hbm, inb, tb, i0, i1, o0, o1):
    i_sems = [i0, i1]
    o_sems = [o0, o1]
    wid = lax.axis_index("s") * _NC + lax.axis_index("c")
    d16k = [lax.iota(jnp.int32, 16) + 16 * k for k in range(_DIM // 16)]

    def in_desc(i, b):
        return pltpu.make_async_copy(
            embt_hbm.at[:, pl.ds(i * _CW, _CW)], inb.at[b], i_sems[b]
        )

    def out_desc(i, b):
        return pltpu.make_async_copy(
            tb.at[b], out_hbm.at[pl.ds(i * (_CW // 2), _CW // 2), :], o_sems[b]
        )

    def do_cols(b):
        inb_b = inb.at[b]
        tb_b = tb.at[b]

        @pl.loop(0, _CW // 2, step=4)
        def pair(r0):
            for dr in range(4):
                r = r0 + dr
                jv0 = jnp.full((16,), 2 * r, jnp.int32)
                jv1 = jv0 + 1
                for k in range(_DIM // 16):
                    tb_b[r, pl.ds(16 * k, 16)] = plsc.load_gather(
                        inb_b, [d16k[k], jv0]
                    )
                    tb_b[r, pl.ds(_DIM + 16 * k, 16)] = plsc.load_gather(
                        inb_b, [d16k[k], jv1]
                    )

    _NT = (_NBLK + _NW - 1) // _NW
    _NSLOT = (_NT + 2 + 1) // 2 * 2

    for b in range(_NBUF):
        i = wid + b * _NW

        @pl.when(i < _NBLK)
        def _():
            in_desc(i, b).start()

    @pl.loop(0, _NSLOT, step=_NBUF)
    def body(g):
        for b in range(_NBUF):
            t = g + b
            i = wid + t * _NW
            prev = i - _NBUF * _NW

            @pl.when(jnp.logical_and(t >= _NBUF, prev < _NBLK))
            def _():
                out_desc(prev, b).wait()

            @pl.when(i < _NBLK)
            def _():
                in_desc(i, b).wait()
                do_cols(b)
                out_desc(i, b).start()
                nxt = i + _NBUF * _NW

                @pl.when(nxt < _NBLK)
                def _():
                    in_desc(nxt, b).start()

    @pl.when(wid == _NBLK % _NW)
    def _():
        c0t = _NBLK * _CW + wid * 0
        pltpu.sync_copy(embt_hbm.at[:, pl.ds(c0t, _CW)], inb.at[0])
        do_cols(0)
        pltpu.sync_copy(
            tb.at[0].at[pl.ds(0, _CW // 4), :],
            out_hbm.at[pl.ds(_NBLK * (_CW // 2), _CW // 4), :],
        )


@functools.partial(
    pl.kernel,
    mesh=_mesh,
    out_type=jax.ShapeDtypeStruct((_B, 2 * _DIM), jnp.float32),
    compiler_params=pltpu.CompilerParams(use_tc_tiling_on_sc=False),
    scratch_types=[
        pltpu.VMEM((_B_PER_W,), jnp.int32),
        pltpu.VMEM((_NBUF, _CHUNK, _DIM), jnp.float32),
        pltpu.SemaphoreType.DMA,
        pltpu.SemaphoreType.DMA,
        pltpu.SemaphoreType.DMA,
        pltpu.SemaphoreType.DMA,
    ],
)
def _emb_lookup(idx_hbm, table_hbm, out_hbm, idx_v, rows_v, g0, g1, o0, o1):
    g_sems = [g0, g1]
    o_sems = [o0, o1]
    wid = lax.axis_index("s") * _NC + lax.axis_index("c")
    base = wid * _B_PER_W

    pltpu.sync_copy(idx_hbm.at[pl.ds(base, _B_PER_W)], idx_v)

    def gather_desc(i, b):
        return pltpu.make_async_copy(
            table_hbm.at[idx_v.at[pl.ds(i * _CHUNK, _CHUNK)]],
            rows_v.at[b],
            g_sems[b],
        )

    def out_desc(i, b):
        return pltpu.make_async_copy(
            rows_v.at[b],
            out_hbm.at[pl.ds(base + i * _CHUNK, _CHUNK), pl.ds(0, _DIM)],
            o_sems[b],
        )

    for b in range(_NBUF):
        gather_desc(b, b).start()

    @pl.loop(0, _N_CHUNKS, step=_NBUF)
    def body(g):
        for b in range(_NBUF):
            i = g + b
            gather_desc(i, b).wait()
            out_desc(i, b).start()
        for b in range(_NBUF):
            i = g + b + _NBUF

            @pl.when(i < _N_CHUNKS)
            def _():
                out_desc(i - _NBUF, b).wait()
                gather_desc(i, b).start()

    for b in range(_NBUF):
        out_desc(_N_CHUNKS - _NBUF + b, b).wait()


def kernel(x, emb_weight):
    pairs = _transpose_k(emb_weight.T)
    table_lin = pairs.reshape(_VOCAB, _DIM)
    flat = x.reshape(_B)
    out2 = _emb_lookup(flat, table_lin)
    return out2[:, :_DIM].reshape(_BATCH, _HIST, _DIM)


# speedup vs baseline: 2.3371x; 1.9621x over previous
import functools

import jax
import jax.numpy as jnp
from jax import lax
from jax.experimental import pallas as pl
from jax.experimental.pallas import tpu as pltpu
from jax.experimental.pallas import tpu_sc as plsc

_VOCAB = 1000000
_DIM = 64
_BATCH = 4096
_HIST = 200

_B = _BATCH * _HIST
_NC = 2
_NS = 16
_NW = _NC * _NS
_B_PER_W = _B // _NW
_CHUNK = 640
_N_CHUNKS = _B_PER_W // _CHUNK
_NBUF = 2

_mesh = plsc.VectorSubcoreMesh(core_axis_name="c", subcore_axis_name="s")


@functools.partial(
    pl.kernel,
    mesh=_mesh,
    out_type=jax.ShapeDtypeStruct((_B, 2 * _DIM), jnp.float32),
    compiler_params=pltpu.CompilerParams(use_tc_tiling_on_sc=False),
    scratch_types=[
        pltpu.VMEM((_B_PER_W,), jnp.int32),
        pltpu.VMEM((_NBUF, _CHUNK, _DIM), jnp.float32),
        pltpu.SemaphoreType.DMA,
        pltpu.SemaphoreType.DMA,
        pltpu.SemaphoreType.DMA,
        pltpu.SemaphoreType.DMA,
    ],
)
def _emb_lookup(idx_hbm, table_hbm, out_hbm, idx_v, rows_v, g0, g1, o0, o1):
    g_sems = [g0, g1]
    o_sems = [o0, o1]
    wid = lax.axis_index("s") * _NC + lax.axis_index("c")
    base = wid * _B_PER_W

    pltpu.sync_copy(idx_hbm.at[pl.ds(base, _B_PER_W)], idx_v)

    def gather_desc(i, b):
        return pltpu.make_async_copy(
            table_hbm.at[idx_v.at[pl.ds(i * _CHUNK, _CHUNK)]],
            rows_v.at[b],
            g_sems[b],
        )

    def out_desc(i, b):
        return pltpu.make_async_copy(
            rows_v.at[b],
            out_hbm.at[pl.ds(base + i * _CHUNK, _CHUNK), pl.ds(0, _DIM)],
            o_sems[b],
        )

    for b in range(_NBUF):
        gather_desc(b, b).start()

    @pl.loop(0, _N_CHUNKS, step=_NBUF)
    def body(g):
        for b in range(_NBUF):
            i = g + b
            gather_desc(i, b).wait()
            out_desc(i, b).start()
        for b in range(_NBUF):
            i = g + b + _NBUF

            @pl.when(i < _N_CHUNKS)
            def _():
                out_desc(i - _NBUF, b).wait()
                gather_desc(i, b).start()

    for b in range(_NBUF):
        out_desc(_N_CHUNKS - _NBUF + b, b).wait()


def kernel(x, emb_weight):
    flat = x.reshape(_B)
    out2 = _emb_lookup(flat, emb_weight)
    return out2[:, :_DIM].reshape(_BATCH, _HIST, _DIM)


# speedup vs baseline: 2.3456x; 1.0036x over previous
import functools

import jax
import jax.numpy as jnp
from jax import lax
from jax.experimental import pallas as pl
from jax.experimental.pallas import tpu as pltpu
from jax.experimental.pallas import tpu_sc as plsc

_VOCAB = 1000000
_DIM = 64
_BATCH = 4096
_HIST = 200

_B = _BATCH * _HIST
_NC = 2
_NS = 16
_NW = _NC * _NS
_B_PER_W = _B // _NW
_CHUNK = 512
_N_CHUNKS = _B_PER_W // _CHUNK
_NBUF = 2

_mesh = plsc.VectorSubcoreMesh(core_axis_name="c", subcore_axis_name="s")


@functools.partial(
    pl.kernel,
    mesh=_mesh,
    out_type=jax.ShapeDtypeStruct((_B, 2 * _DIM), jnp.float32),
    compiler_params=pltpu.CompilerParams(use_tc_tiling_on_sc=False),
    scratch_types=[
        pltpu.VMEM((_B_PER_W,), jnp.int32),
        pltpu.VMEM((_NBUF, _CHUNK, _DIM), jnp.float32),
        pltpu.SemaphoreType.DMA,
        pltpu.SemaphoreType.DMA,
        pltpu.SemaphoreType.DMA,
        pltpu.SemaphoreType.DMA,
    ],
)
def _emb_lookup(idx_hbm, table_hbm, out_hbm, idx_v, rows_v, g0, g1, o0, o1):
    g_sems = [g0, g1]
    o_sems = [o0, o1]
    wid = lax.axis_index("s") * _NC + lax.axis_index("c")
    base = wid * _B_PER_W

    pltpu.sync_copy(idx_hbm.at[pl.ds(base, _B_PER_W)], idx_v)

    def gather_desc(i, b):
        return pltpu.make_async_copy(
            table_hbm.at[idx_v.at[pl.ds(i * _CHUNK, _CHUNK)]],
            rows_v.at[b],
            g_sems[b],
        )

    def out_desc(i, b):
        return pltpu.make_async_copy(
            rows_v.at[b],
            out_hbm.at[pl.ds(base + i * _CHUNK, _CHUNK), pl.ds(0, _DIM)],
            o_sems[b],
        )

    for b in range(_NBUF):
        gather_desc(b, b).start()

    @pl.loop(0, _N_CHUNKS, step=_NBUF)
    def body(g):
        for b in range(_NBUF):
            i = g + b
            gather_desc(i, b).wait()
            out_desc(i, b).start()
        for b in range(_NBUF):
            i = g + b + _NBUF

            @pl.when(i < _N_CHUNKS)
            def _():
                out_desc(i - _NBUF, b).wait()
                gather_desc(i, b).start()

    for b in range(_NBUF):
        out_desc(_N_CHUNKS - _NBUF + b, b).wait()


def kernel(x, emb_weight):
    flat = x.reshape(_B)
    out2 = _emb_lookup(flat, emb_weight)
    return out2[:, :_DIM].reshape(_BATCH, _HIST, _DIM)
